# Initial kernel scaffold; baseline (speedup 1.0000x reference)
import jax
import jax.numpy as jnp
from jax.experimental import pallas as pl


def kernel(node_attr, edge_index, edge_attr, edge_sh, Wh, bh, Wo, bo):
    raise NotImplementedError("write your pallas kernel here")


# speedup vs baseline: 3.1000x; 3.1000x over previous
import functools

import numpy as np
import jax
import jax.numpy as jnp
from jax import lax
from jax.experimental import pallas as pl
from jax.experimental.pallas import tpu as pltpu
from jax.experimental.pallas import tpu_sc as plsc

MUL = 16
ALPHA = 1.0 / np.sqrt(2.0 * MUL)
INV_SQRT3 = 1.0 / np.sqrt(3.0)

_SC_CORES = 2
_SC_SUBCORES = 16
_WINDOW = 128
_SWINDOW = 128
_LANES = 128

_PERM_IN = np.array(
    list(range(16)) + [16 + u * 3 + i for i in range(3) for u in range(16)],
    dtype=np.int32)

_S16_NP = np.zeros((256, 16), np.float32)
for _w in range(16):
    for _u in range(16):
        _S16_NP[_w * 16 + _u, _w] = 1.0

_POUT_NP = np.zeros((64, 64), np.float32)
for _w in range(16):
    _POUT_NP[_w, _w] = 1.0
for _i in range(3):
    for _w in range(16):
        _POUT_NP[16 + _i * 16 + _w, 16 + _w * 3 + _i] = 1.0


def _vector_mesh():
    return plsc.VectorSubcoreMesh(core_axis_name="core", subcore_axis_name="subcore")


def _sc_gather(node_p, src2d):
    n_edges = src2d.shape[1]
    grid = n_edges // _WINDOW

    @functools.partial(
        pl.kernel,
        out_type=jax.ShapeDtypeStruct((n_edges, _LANES), node_p.dtype),
        mesh=_vector_mesh(),
    )
    def gk(node_hbm, idx_hbm, out_hbm):
        def body(i_v, o_v):
            pltpu.sync_copy(node_hbm.at[i_v.at[0]], o_v)

        pltpu.emit_pipeline(
            body,
            grid=(grid,),
            in_specs=[pl.BlockSpec((1, _WINDOW), lambda i: (0, i))],
            out_specs=[pl.BlockSpec((_WINDOW, _LANES), lambda i: (i, 0))],
            core_axis_name=("core", "subcore"),
            dimension_semantics=(pltpu.PARALLEL,),
        )(idx_hbm, out_hbm)

    return gk(node_p, src2d)


def _sc_scatter(msg128, dst2d, src2d, zeros_hbm, n_nodes):
    n_edges = msg128.shape[0]
    grid = n_edges // _SWINDOW
    big = 640
    rest = n_nodes - 15 * big
    nz = 80

    @functools.partial(
        pl.kernel,
        out_type=jax.ShapeDtypeStruct((_SC_CORES * n_nodes, _LANES),
                                      jnp.float32),
        mesh=_vector_mesh(),
        scratch_types=[
            pltpu.VMEM_SHARED((n_nodes, _LANES), jnp.float32),
            pltpu.VMEM((_SWINDOW, _LANES), jnp.float32),
        ],
    )
    def sk(msg_hbm, dst_hbm, srcidx_hbm, z_hbm, aggp_hbm, agg_sh, ones_v):
        c = lax.axis_index("core")
        s = lax.axis_index("subcore")
        zrow = jnp.zeros((16,), jnp.float32)
        onerow = jnp.where(lax.iota(jnp.int32, 16) == 0, 1.0, 0.0)

        @pl.loop(0, _SWINDOW)
        def _(i):
            for k in range(_LANES // 16):
                ones_v[i, pl.ds(16 * k, 16)] = onerow if k == 4 else zrow

        @pl.when(s < 15)
        def _():
            @pl.loop(0, big // nz)
            def _(k):
                pltpu.sync_copy(z_hbm, agg_sh.at[pl.ds(s * big + k * nz, nz)])

        @pl.when(s == 15)
        def _():
            @pl.loop(0, rest // nz)
            def _(k):
                pltpu.sync_copy(z_hbm, agg_sh.at[pl.ds(15 * big + k * nz, nz)])

        plsc.subcore_barrier()

        def body(m_v, d_v, si_v):
            pltpu.sync_copy(m_v, agg_sh.at[d_v.at[0]], add=True)
            pltpu.sync_copy(ones_v, agg_sh.at[si_v.at[0]], add=True)

        pltpu.emit_pipeline(
            body,
            grid=(grid,),
            in_specs=[
                pl.BlockSpec((_SWINDOW, _LANES), lambda i: (i, 0)),
                pl.BlockSpec((1, _SWINDOW), lambda i: (0, i)),
                pl.BlockSpec((1, _SWINDOW), lambda i: (0, i)),
            ],
            out_specs=[],
            core_axis_name=("core", "subcore"),
            dimension_semantics=(pltpu.PARALLEL,),
        )(msg_hbm, dst_hbm, srcidx_hbm)

        plsc.subcore_barrier()

        @pl.when(s < 15)
        def _():
            pltpu.sync_copy(agg_sh.at[pl.ds(s * big, big)],
                            aggp_hbm.at[pl.ds(c * n_nodes + s * big, big)])

        @pl.when(s == 15)
        def _():
            pltpu.sync_copy(agg_sh.at[pl.ds(15 * big, rest)],
                            aggp_hbm.at[pl.ds(c * n_nodes + 15 * big, rest)])

    return sk(msg128, dst2d, src2d, zeros_hbm)


def _tc_dense(xg, ea, sh, Wh, bh2, Wbig, bbig2, s16, block):
    n_edges = xg.shape[0]
    grid = n_edges // block

    def body(x_ref, ea_ref, sh_ref, wh_ref, bh_ref, wb_ref, bb_ref, s16_ref,
             msg_ref):
        x = x_ref[...]
        shv = sh_ref[...]
        hp = jnp.dot(ea_ref[...], wh_ref[...],
                     preferred_element_type=jnp.float32) + bh_ref[...]
        h = hp * (1.0 / (1.0 + jnp.exp(-hp)))
        wb = jnp.dot(h, wb_ref[...],
                     preferred_element_type=jnp.float32) + bb_ref[...]
        x0 = x[:, :16]
        x1 = [x[:, 16 + 16 * i:32 + 16 * i] for i in range(3)]
        y0 = shv[:, 0:1]
        y1 = [shv[:, 1 + i:2 + i] for i in range(3)]
        a1 = x0 * y0
        dotv = (x1[0] * y1[0] + x1[1] * y1[1] + x1[2] * y1[2]) * INV_SQRT3
        vecs = [a1, dotv, x0, x1[0] * y0, x1[1] * y0, x1[2] * y0]
        vbig = jnp.concatenate(
            [jnp.concatenate([v] * 16, axis=1) for v in vecs], axis=1)
        p = wb * vbig
        p01 = p[:, :256] + p[:, 256:512]
        pstk = jnp.concatenate(
            [p01, p[:, 512:768], p[:, 768:1024], p[:, 1024:1280],
             p[:, 1280:1536]], axis=0)
        t = jnp.dot(pstk, s16_ref[...], preferred_element_type=jnp.float32)
        out0 = t[:block]
        t2 = t[block:2 * block]
        out1 = [t2 * y1[i] + t[(2 + i) * block:(3 + i) * block]
                for i in range(3)]
        msg = jnp.concatenate(
            [out0, out1[0], out1[1], out1[2]], axis=1) * ALPHA
        msg_ref[...] = jnp.concatenate(
            [msg, jnp.zeros((block, _LANES - 64), jnp.float32)], axis=1)

    return pl.pallas_call(
        body,
        grid=(grid,),
        in_specs=[
            pl.BlockSpec((block, _LANES), lambda i: (i, 0)),
            pl.BlockSpec((block, 16), lambda i: (i, 0)),
            pl.BlockSpec((block, 4), lambda i: (i, 0)),
            pl.BlockSpec((16, 16), lambda i: (0, 0)),
            pl.BlockSpec((1, 16), lambda i: (0, 0)),
            pl.BlockSpec((16, 1536), lambda i: (0, 0)),
            pl.BlockSpec((1, 1536), lambda i: (0, 0)),
            pl.BlockSpec((256, 16), lambda i: (0, 0)),
        ],
        out_specs=pl.BlockSpec((block, _LANES), lambda i: (i, 0)),
        out_shape=jax.ShapeDtypeStruct((n_edges, _LANES), jnp.float32),
        compiler_params=pltpu.CompilerParams(
            dimension_semantics=("arbitrary",)),
    )(xg, ea, sh, Wh, bh2, Wbig, bbig2, s16)


def _tc_finalize(aggp, pout, n_nodes):
    def body(a_ref, p_ref, o_ref):
        a = a_ref[...]
        agg = a[:n_nodes, :64] + a[n_nodes:, :64]
        deg = a[:n_nodes, 64:65] + a[n_nodes:, 64:65]
        res = agg / jnp.maximum(deg, 1.0)
        o_ref[...] = jnp.dot(res, p_ref[...],
                             preferred_element_type=jnp.float32)

    return pl.pallas_call(
        body,
        out_shape=jax.ShapeDtypeStruct((n_nodes, 64), jnp.float32),
    )(aggp, pout)


def kernel(node_attr, edge_index, edge_attr, edge_sh, Wh, bh, Wo, bo):
    n_nodes, feat = node_attr.shape
    n_edges = edge_index.shape[1]

    node_p = jnp.pad(node_attr[:, _PERM_IN], ((0, 0), (0, _LANES - feat)))
    src2d = edge_index[0].reshape(1, n_edges)
    dst2d = edge_index[1].reshape(1, n_edges)
    wo4 = Wo.reshape(16, 4, 16, 16)
    wp = [jnp.swapaxes(wo4[:, p], 1, 2).reshape(16, 256) for p in range(4)]
    wbig = jnp.concatenate([wp[0], wp[3], wp[1], wp[2], wp[2], wp[2]], axis=1)
    bo4 = bo.reshape(4, 16, 16)
    bp = [bo4[p].T.reshape(256) for p in range(4)]
    bbig2 = jnp.concatenate(
        [bp[0], bp[3], bp[1], bp[2], bp[2], bp[2]]).reshape(1, 1536)
    bh2 = bh.reshape(1, 16)
    s16 = jnp.asarray(_S16_NP)
    pout = jnp.asarray(_POUT_NP)

    xg = _sc_gather(node_p, src2d)
    msg128 = _tc_dense(xg, edge_attr, edge_sh, Wh, bh2, wbig, bbig2, s16,
                       block=1000)
    zeros_hbm = jnp.zeros((80, _LANES), jnp.float32)
    aggp = _sc_scatter(msg128, dst2d, src2d, zeros_hbm, n_nodes)
    return _tc_finalize(aggp, pout, n_nodes)
